# Initial kernel scaffold; baseline (speedup 1.0000x reference)
import jax
import jax.numpy as jnp
from jax.experimental import pallas as pl


def kernel(img):
    raise NotImplementedError("write your pallas kernel here")


# speedup vs baseline: 83.5401x; 83.5401x over previous
import math

import jax
import jax.numpy as jnp
from jax.experimental import pallas as pl

_NUM_BINS = 9
_CELL = 8
_PI = math.pi

_ATAN_C = (
    0.99997726,
    -0.33262347,
    0.19354346,
    -0.11643287,
    0.05265332,
    -0.01172120,
)


def _atan01(a):
    z = a * a
    p = jnp.float32(_ATAN_C[5])
    for c in _ATAN_C[4::-1]:
        p = p * z + jnp.float32(c)
    return a * p


def _hog_body(x_ref, o_ref):
    x = x_ref[0]
    C, H, W = x.shape
    nHc, nWc = H // _CELL, W // _CELL

    zc = jnp.zeros((C, H, 1), jnp.float32)
    gx = jnp.concatenate([zc, x[:, :, 2:] - x[:, :, :-2], zc], axis=2)
    zr = jnp.zeros((C, 1, W), jnp.float32)
    gy = jnp.concatenate([zr, x[:, 2:, :] - x[:, :-2, :], zr], axis=1)

    m = gx * gx + gy * gy + 1e-12
    m0, m1, m2 = m[0], m[1], m[2]
    c0 = (m0 >= m1) & (m0 >= m2)
    c1 = jnp.logical_and(~c0, m1 >= m2)
    gxs = jnp.where(c0, gx[0], jnp.where(c1, gx[1], gx[2]))
    gys = jnp.where(c0, gy[0], jnp.where(c1, gy[1], gy[2]))
    mag = jnp.sqrt(jnp.where(c0, m0, jnp.where(c1, m1, m2)))

    ax = jnp.abs(gxs)
    ay = jnp.abs(gys)
    mn = jnp.minimum(ax, ay)
    mx = jnp.maximum(ax, ay)
    r = _atan01(mn / jnp.maximum(mx, 1e-30))
    phi = jnp.where(ay > ax, jnp.float32(0.5 * _PI) - r, r)
    neg = (gxs < 0) != (gys < 0)
    theta = jnp.where(neg, jnp.float32(_PI) - phi, phi)

    b = theta * jnp.float32(_NUM_BINS / _PI) - 0.5
    b0 = jnp.floor(b)
    w1 = b - b0
    v1 = mag * w1
    v0 = mag - v1
    k0 = jnp.where(b0 < 0, jnp.float32(_NUM_BINS - 1), b0)
    k1 = jnp.where(k0 >= _NUM_BINS - 1, jnp.float32(0.0), k0 + 1.0)

    ri = jax.lax.broadcasted_iota(jnp.int32, (W, nWc), 0) // _CELL
    ci = jax.lax.broadcasted_iota(jnp.int32, (W, nWc), 1)
    P = (ri == ci).astype(jnp.float32)

    rows = []
    for k in range(_NUM_BINS):
        kf = jnp.float32(k)
        ck = jnp.where(k0 == kf, v0, 0.0) + jnp.where(k1 == kf, v1, 0.0)
        rows.append(ck.reshape(nHc, _CELL, W).sum(axis=1))
    A = jnp.concatenate(rows, axis=0)
    hog = jnp.dot(A, P, preferred_element_type=jnp.float32)
    hog = hog.reshape(_NUM_BINS, nHc, nWc)

    inv = jax.lax.rsqrt(jnp.sum(hog * hog, axis=0) + 1e-6)
    o_ref[0] = hog * inv


def kernel(img):
    B, C, H, W = img.shape
    nHc, nWc = H // _CELL, W // _CELL
    return pl.pallas_call(
        _hog_body,
        grid=(B,),
        in_specs=[pl.BlockSpec((1, C, H, W), lambda b: (b, 0, 0, 0))],
        out_specs=pl.BlockSpec((1, _NUM_BINS, nHc, nWc), lambda b: (b, 0, 0, 0)),
        out_shape=jax.ShapeDtypeStruct((B, _NUM_BINS, nHc, nWc), jnp.float32),
    )(img)
